# Initial kernel scaffold; baseline (speedup 1.0000x reference)
import jax
import jax.numpy as jnp
from jax.experimental import pallas as pl


def kernel(x, table, W, b):
    raise NotImplementedError("write your pallas kernel here")


# speedup vs baseline: 1.2225x; 1.2225x over previous
import functools

import jax
import jax.numpy as jnp
from jax import lax
from jax.experimental import pallas as pl
from jax.experimental.pallas import tpu as pltpu
from jax.experimental.pallas import tpu_sc as plsc

_VOCAB = 256
_BATCH = 4096
_HIST = 50
_NC, _NS = 2, 16
_NW = _NC * _NS
_TOTAL = _BATCH * _HIST
_PER_W = _TOTAL // _NW
_CH = 128
_NCHUNK = _PER_W // _CH


def _logits_body(table_ref, w_ref, b_ref, out_ref):
    out_ref[...] = lax.dot_general(
        table_ref[...], w_ref[...], (((1,), (1,)), ((), ())),
        preferred_element_type=jnp.float32) + b_ref[...]


def _compute_logits(table, W, b):
    return pl.pallas_call(
        _logits_body,
        out_shape=jax.ShapeDtypeStruct((_VOCAB, _VOCAB), jnp.float32),
    )(table, W, b.reshape(1, _VOCAB))


@functools.partial(
    pl.kernel,
    mesh=plsc.VectorSubcoreMesh(core_axis_name="c", subcore_axis_name="s"),
    out_type=jax.ShapeDtypeStruct((_TOTAL, _VOCAB), jnp.float32),
    scratch_types=[
        pltpu.VMEM((_NCHUNK, _CH), jnp.int32),
        pltpu.VMEM((_CH, _VOCAB), jnp.float32),
        pltpu.SemaphoreType.DMA,
    ],
)
def _sc_gather(x_hbm, logits_hbm, out_hbm, idx_v, rows_v, gsem):
    wid = lax.axis_index("s") * _NC + lax.axis_index("c")
    pltpu.sync_copy(x_hbm.at[wid], idx_v)
    base0 = wid * _PER_W

    def body(i, carry):
        pltpu.async_copy(logits_hbm.at[idx_v.at[i]], rows_v, gsem).wait()
        pltpu.sync_copy(rows_v, out_hbm.at[pl.ds(base0 + i * _CH, _CH)])
        return carry

    lax.fori_loop(0, _NCHUNK, body, 0)


def kernel(x, table, W, b):
    logits = _compute_logits(table, W, b)
    xf = x.reshape(_TOTAL).astype(jnp.int32).reshape(_NW, _NCHUNK, _CH)
    out = _sc_gather(xf, logits)
    return out.reshape(_BATCH, _HIST, _VOCAB)
